# initial kernel scaffold (unmeasured)
import jax
import jax.numpy as jnp
from jax import lax
from jax.experimental import pallas as pl
from jax.experimental.pallas import tpu as pltpu

N_DEV = 4
N_HOPS = 2 * (N_DEV - 1)


def kernel(x, k, Wp):
    B, S, C = x.shape
    P = Wp.shape[1]

    def body(x_ref, k_ref, w_ref, out_ref, comm_ref, send_sems, recv_sems,
             credit_sem):
        my = lax.axis_index("i")
        left = lax.rem(my + N_DEV - 1, N_DEV)
        right = lax.rem(my + 1, N_DEV)

        w = w_ref[...].astype(jnp.bfloat16)
        for b in range(B):
            xb = x_ref[b]
            pad = jnp.concatenate(
                [jnp.zeros((3, C), jnp.float32), xb], axis=0)
            conv = (pad[0:S] * k_ref[0]
                    + pad[1:S + 1] * k_ref[1]
                    + pad[2:S + 2] * k_ref[2]
                    + pad[3:S + 3] * k_ref[3])
            a = (conv * jax.nn.sigmoid(conv)).astype(jnp.bfloat16)
            y = jnp.dot(a, w, preferred_element_type=jnp.float32)
            out_ref[b] = y.astype(jnp.bfloat16)

        bar = pltpu.get_barrier_semaphore()
        for nbr in (left, right):
            pl.semaphore_signal(bar, inc=1, device_id=(nbr,),
                                device_id_type=pl.DeviceIdType.MESH)
        pl.semaphore_wait(bar, 2)

        for h in range(N_HOPS):
            if h < N_DEV - 1:
                off = -h
            else:
                off = 1 - (h - (N_DEV - 1))
            send_c = lax.rem(my + off + 2 * N_DEV, N_DEV)
            recv_c = lax.rem(send_c + N_DEV - 1, N_DEV)
            slot = h % 2

            if h >= 2:
                pl.semaphore_wait(credit_sem, 1)

            rdma = pltpu.make_async_remote_copy(
                src_ref=out_ref.at[send_c],
                dst_ref=comm_ref.at[slot],
                send_sem=send_sems.at[slot],
                recv_sem=recv_sems.at[slot],
                device_id=(right,),
                device_id_type=pl.DeviceIdType.MESH,
            )
            rdma.start()
            rdma.wait()

            if h < N_DEV - 1:
                out_ref[recv_c] = out_ref[recv_c] + comm_ref[slot]
            else:
                out_ref[recv_c] = comm_ref[slot]

            if h <= N_HOPS - 3:
                pl.semaphore_signal(credit_sem, inc=1, device_id=(left,),
                                    device_id_type=pl.DeviceIdType.MESH)

    out = pl.pallas_call(
        body,
        out_shape=jax.ShapeDtypeStruct((B, S, P), jnp.bfloat16),
        in_specs=[
            pl.BlockSpec(memory_space=pltpu.VMEM),
            pl.BlockSpec(memory_space=pltpu.VMEM),
            pl.BlockSpec(memory_space=pltpu.VMEM),
        ],
        out_specs=pl.BlockSpec(memory_space=pltpu.VMEM),
        scratch_shapes=[
            pltpu.VMEM((2, S, P), jnp.bfloat16),
            pltpu.SemaphoreType.DMA((2,)),
            pltpu.SemaphoreType.DMA((2,)),
            pltpu.SemaphoreType.REGULAR,
        ],
        compiler_params=pltpu.CompilerParams(collective_id=0),
    )(x, k, Wp)
    return out.astype(jnp.float32)


# baseline (device time: 375547 ns/iter reference)
import jax
import jax.numpy as jnp
from jax import lax
from jax.experimental import pallas as pl
from jax.experimental.pallas import tpu as pltpu

N_DEV = 4
N_HOPS = 2 * (N_DEV - 1)


def kernel(x, k, Wp):
    B, S, C = x.shape
    P = Wp.shape[1]

    def body(x_ref, k_ref, w_ref, out_ref, comm_ref, send_sems, recv_sems,
             credit_sem):
        my = lax.axis_index("i")
        left = lax.rem(my + N_DEV - 1, N_DEV)
        right = lax.rem(my + 1, N_DEV)

        w = w_ref[...]
        for b in range(B):
            xb = x_ref[b]
            pad = jnp.concatenate(
                [jnp.zeros((3, C), jnp.bfloat16), xb], axis=0)
            conv = (pad[0:S] * k_ref[0]
                    + pad[1:S + 1] * k_ref[1]
                    + pad[2:S + 2] * k_ref[2]
                    + pad[3:S + 3] * k_ref[3])
            a = (conv * jax.nn.sigmoid(conv)).astype(jnp.bfloat16)
            y = jnp.dot(a, w, preferred_element_type=jnp.float32)
            out_ref[b] = y.astype(jnp.bfloat16)

        bar = pltpu.get_barrier_semaphore()
        for nbr in (left, right):
            pl.semaphore_signal(bar, inc=1, device_id=(nbr,),
                                device_id_type=pl.DeviceIdType.MESH)
        pl.semaphore_wait(bar, 2)

        for h in range(N_HOPS):
            if h < N_DEV - 1:
                off = -h
            else:
                off = 1 - (h - (N_DEV - 1))
            send_c = lax.rem(my + off + 2 * N_DEV, N_DEV)
            recv_c = lax.rem(send_c + N_DEV - 1, N_DEV)
            slot = h % 2

            if h >= 2:
                pl.semaphore_wait(credit_sem, 1)

            rdma = pltpu.make_async_remote_copy(
                src_ref=out_ref.at[send_c],
                dst_ref=comm_ref.at[slot],
                send_sem=send_sems.at[slot],
                recv_sem=recv_sems.at[slot],
                device_id=(right,),
                device_id_type=pl.DeviceIdType.MESH,
            )
            rdma.start()
            rdma.wait()

            if h < N_DEV - 1:
                out_ref[recv_c] = out_ref[recv_c] + comm_ref[slot]
            else:
                out_ref[recv_c] = comm_ref[slot]

            if h <= N_HOPS - 3:
                pl.semaphore_signal(credit_sem, inc=1, device_id=(left,),
                                    device_id_type=pl.DeviceIdType.MESH)

    out = pl.pallas_call(
        body,
        out_shape=jax.ShapeDtypeStruct((B, S, P), jnp.bfloat16),
        in_specs=[
            pl.BlockSpec(memory_space=pltpu.VMEM),
            pl.BlockSpec(memory_space=pltpu.VMEM),
            pl.BlockSpec(memory_space=pltpu.VMEM),
        ],
        out_specs=pl.BlockSpec(memory_space=pltpu.VMEM),
        scratch_shapes=[
            pltpu.VMEM((2, S, P), jnp.bfloat16),
            pltpu.SemaphoreType.DMA((2,)),
            pltpu.SemaphoreType.DMA((2,)),
            pltpu.SemaphoreType.REGULAR,
        ],
        compiler_params=pltpu.CompilerParams(
            collective_id=0, vmem_limit_bytes=100 * 1024 * 1024),
    )(x.astype(jnp.bfloat16), k, Wp.astype(jnp.bfloat16))
    return out.astype(jnp.float32)


# device time: 240300 ns/iter; 1.5628x vs baseline; 1.5628x over previous
import jax
import jax.numpy as jnp
from jax import lax
from jax.experimental import pallas as pl
from jax.experimental.pallas import tpu as pltpu

N_DEV = 4
N_HOPS = 2 * (N_DEV - 1)


def kernel(x, k, Wp):
    B, S, C = x.shape
    P = Wp.shape[1]
    H = S // 2

    def body(x_ref, k_ref, w_ref, out_ref, comm_r, comm_l,
             send_sems, recv_sems, credit_sems):
        my = lax.axis_index("i")
        left = lax.rem(my + N_DEV - 1, N_DEV)
        right = lax.rem(my + 1, N_DEV)

        w = w_ref[...]
        for b in range(B):
            xb = x_ref[b]
            pad = jnp.concatenate(
                [jnp.zeros((3, C), jnp.bfloat16), xb], axis=0)
            conv = (pad[0:S] * k_ref[0]
                    + pad[1:S + 1] * k_ref[1]
                    + pad[2:S + 2] * k_ref[2]
                    + pad[3:S + 3] * k_ref[3])
            a = (conv * jax.nn.sigmoid(conv)).astype(jnp.bfloat16)
            y = jnp.dot(a, w, preferred_element_type=jnp.float32)
            out_ref[b] = y.astype(jnp.bfloat16)

        bar = pltpu.get_barrier_semaphore()
        for nbr in (left, right):
            pl.semaphore_signal(bar, inc=1, device_id=(nbr,),
                                device_id_type=pl.DeviceIdType.MESH)
        pl.semaphore_wait(bar, 2)

        for h in range(N_HOPS):
            if h < N_DEV - 1:
                off_r, off_l = -h, h
            else:
                off_r, off_l = 4 - h, h - 4
            send_r = lax.rem(my + off_r + 2 * N_DEV, N_DEV)
            send_l = lax.rem(my + off_l + 2 * N_DEV, N_DEV)
            recv_r = lax.rem(send_r + N_DEV - 1, N_DEV)
            recv_l = lax.rem(send_l + 1, N_DEV)
            slot = h % 2

            if h >= 2:
                pl.semaphore_wait(credit_sems.at[0], 1)
                pl.semaphore_wait(credit_sems.at[1], 1)

            rdma_r = pltpu.make_async_remote_copy(
                src_ref=out_ref.at[send_r, pl.ds(0, H)],
                dst_ref=comm_r.at[slot],
                send_sem=send_sems.at[0, slot],
                recv_sem=recv_sems.at[0, slot],
                device_id=(right,),
                device_id_type=pl.DeviceIdType.MESH,
            )
            rdma_l = pltpu.make_async_remote_copy(
                src_ref=out_ref.at[send_l, pl.ds(H, H)],
                dst_ref=comm_l.at[slot],
                send_sem=send_sems.at[1, slot],
                recv_sem=recv_sems.at[1, slot],
                device_id=(left,),
                device_id_type=pl.DeviceIdType.MESH,
            )
            rdma_r.start()
            rdma_l.start()
            rdma_r.wait()
            rdma_l.wait()

            if h < N_DEV - 1:
                out_ref[recv_r, pl.ds(0, H)] = (
                    out_ref[recv_r, pl.ds(0, H)] + comm_r[slot])
                out_ref[recv_l, pl.ds(H, H)] = (
                    out_ref[recv_l, pl.ds(H, H)] + comm_l[slot])
            else:
                out_ref[recv_r, pl.ds(0, H)] = comm_r[slot]
                out_ref[recv_l, pl.ds(H, H)] = comm_l[slot]

            if h <= N_HOPS - 3:
                pl.semaphore_signal(credit_sems.at[0], inc=1,
                                    device_id=(left,),
                                    device_id_type=pl.DeviceIdType.MESH)
                pl.semaphore_signal(credit_sems.at[1], inc=1,
                                    device_id=(right,),
                                    device_id_type=pl.DeviceIdType.MESH)

    out = pl.pallas_call(
        body,
        out_shape=jax.ShapeDtypeStruct((B, S, P), jnp.bfloat16),
        in_specs=[
            pl.BlockSpec(memory_space=pltpu.VMEM),
            pl.BlockSpec(memory_space=pltpu.VMEM),
            pl.BlockSpec(memory_space=pltpu.VMEM),
        ],
        out_specs=pl.BlockSpec(memory_space=pltpu.VMEM),
        scratch_shapes=[
            pltpu.VMEM((2, H, P), jnp.bfloat16),
            pltpu.VMEM((2, H, P), jnp.bfloat16),
            pltpu.SemaphoreType.DMA((2, 2)),
            pltpu.SemaphoreType.DMA((2, 2)),
            pltpu.SemaphoreType.REGULAR((2,)),
        ],
        compiler_params=pltpu.CompilerParams(
            collective_id=0, vmem_limit_bytes=100 * 1024 * 1024),
    )(x.astype(jnp.bfloat16), k, Wp.astype(jnp.bfloat16))
    return out.astype(jnp.float32)


# device time: 211563 ns/iter; 1.7751x vs baseline; 1.1358x over previous
import jax
import jax.numpy as jnp
from jax import lax
from jax.experimental import pallas as pl
from jax.experimental.pallas import tpu as pltpu

N_DEV = 4
N_HOPS = 2 * (N_DEV - 1)


def kernel(x, k, Wp):
    B, S, C = x.shape
    P = Wp.shape[1]
    H = S // 2

    def body(x_ref, k_ref, w_ref, out_ref, comm_r, comm_l,
             send_sems, recv_sems, credit_sems):
        my = lax.axis_index("i")
        left = lax.rem(my + N_DEV - 1, N_DEV)
        right = lax.rem(my + 1, N_DEV)
        cp1 = right
        cm1 = left
        cp2 = lax.rem(my + 2, N_DEV)

        w = w_ref[...]

        def compute_half(c, half):
            if half == 0:
                seg = jnp.concatenate(
                    [jnp.zeros((3, C), jnp.bfloat16), x_ref[c, pl.ds(0, H)]],
                    axis=0)
            else:
                seg = x_ref[c, pl.ds(H - 3, H + 3)]
            conv = (seg[0:H] * k_ref[0]
                    + seg[1:H + 1] * k_ref[1]
                    + seg[2:H + 2] * k_ref[2]
                    + seg[3:H + 3] * k_ref[3])
            a = (conv * jax.nn.sigmoid(conv)).astype(jnp.bfloat16)
            y = jnp.dot(a, w, preferred_element_type=jnp.float32)
            out_ref[c, pl.ds(half * H, H)] = y.astype(jnp.bfloat16)

        compute_half(my, 0)
        compute_half(my, 1)

        bar = pltpu.get_barrier_semaphore()
        for nbr in (left, right):
            pl.semaphore_signal(bar, inc=1, device_id=(nbr,),
                                device_id_type=pl.DeviceIdType.MESH)
        pl.semaphore_wait(bar, 2)

        def hop(h, overlapped_compute=None):
            if h < N_DEV - 1:
                off_r, off_l = -h, h
            else:
                off_r, off_l = 4 - h, h - 4
            send_r = lax.rem(my + off_r + 2 * N_DEV, N_DEV)
            send_l = lax.rem(my + off_l + 2 * N_DEV, N_DEV)
            recv_r = lax.rem(send_r + N_DEV - 1, N_DEV)
            recv_l = lax.rem(send_l + 1, N_DEV)
            slot = h % 2

            if h >= 2:
                pl.semaphore_wait(credit_sems.at[0], 1)
                pl.semaphore_wait(credit_sems.at[1], 1)

            rdma_r = pltpu.make_async_remote_copy(
                src_ref=out_ref.at[send_r, pl.ds(0, H)],
                dst_ref=comm_r.at[slot],
                send_sem=send_sems.at[0, slot],
                recv_sem=recv_sems.at[0, slot],
                device_id=(right,),
                device_id_type=pl.DeviceIdType.MESH,
            )
            rdma_l = pltpu.make_async_remote_copy(
                src_ref=out_ref.at[send_l, pl.ds(H, H)],
                dst_ref=comm_l.at[slot],
                send_sem=send_sems.at[1, slot],
                recv_sem=recv_sems.at[1, slot],
                device_id=(left,),
                device_id_type=pl.DeviceIdType.MESH,
            )
            rdma_r.start()
            rdma_l.start()

            if overlapped_compute is not None:
                overlapped_compute()

            rdma_r.wait()
            rdma_l.wait()

            if h < N_DEV - 1:
                out_ref[recv_r, pl.ds(0, H)] = (
                    out_ref[recv_r, pl.ds(0, H)] + comm_r[slot])
                out_ref[recv_l, pl.ds(H, H)] = (
                    out_ref[recv_l, pl.ds(H, H)] + comm_l[slot])
            else:
                out_ref[recv_r, pl.ds(0, H)] = comm_r[slot]
                out_ref[recv_l, pl.ds(H, H)] = comm_l[slot]

            if h <= N_HOPS - 3:
                pl.semaphore_signal(credit_sems.at[0], inc=1,
                                    device_id=(left,),
                                    device_id_type=pl.DeviceIdType.MESH)
                pl.semaphore_signal(credit_sems.at[1], inc=1,
                                    device_id=(right,),
                                    device_id_type=pl.DeviceIdType.MESH)

        hop(0, lambda: (compute_half(cm1, 0), compute_half(cp1, 1)))
        hop(1, lambda: (compute_half(cp2, 0), compute_half(cp2, 1)))
        hop(2, lambda: (compute_half(cp1, 0), compute_half(cm1, 1)))
        hop(3)
        hop(4)
        hop(5)

    out = pl.pallas_call(
        body,
        out_shape=jax.ShapeDtypeStruct((B, S, P), jnp.bfloat16),
        in_specs=[
            pl.BlockSpec(memory_space=pltpu.VMEM),
            pl.BlockSpec(memory_space=pltpu.VMEM),
            pl.BlockSpec(memory_space=pltpu.VMEM),
        ],
        out_specs=pl.BlockSpec(memory_space=pltpu.VMEM),
        scratch_shapes=[
            pltpu.VMEM((2, H, P), jnp.bfloat16),
            pltpu.VMEM((2, H, P), jnp.bfloat16),
            pltpu.SemaphoreType.DMA((2, 2)),
            pltpu.SemaphoreType.DMA((2, 2)),
            pltpu.SemaphoreType.REGULAR((2,)),
        ],
        compiler_params=pltpu.CompilerParams(
            collective_id=0, vmem_limit_bytes=100 * 1024 * 1024),
    )(x.astype(jnp.bfloat16), k, Wp.astype(jnp.bfloat16))
    return out.astype(jnp.float32)


# device time: 206585 ns/iter; 1.8179x vs baseline; 1.0241x over previous
import jax
import jax.numpy as jnp
from jax import lax
from jax.experimental import pallas as pl
from jax.experimental.pallas import tpu as pltpu

N_DEV = 4
N_HOPS = 2 * (N_DEV - 1)


def kernel(x, k, Wp):
    B, S, C = x.shape
    P = Wp.shape[1]
    H = S // 2

    def body(x_ref, k_ref, w_ref, out_ref, comm_r, comm_l,
             send_sems, recv_sems, credit_sems):
        my = lax.axis_index("i")
        left = lax.rem(my + N_DEV - 1, N_DEV)
        right = lax.rem(my + 1, N_DEV)
        cp1 = right
        cm1 = left
        cp2 = lax.rem(my + 2, N_DEV)

        w = w_ref[...]

        def compute_half(c, half):
            if half == 0:
                seg = jnp.concatenate(
                    [jnp.zeros((3, C), jnp.bfloat16), x_ref[c, pl.ds(0, H)]],
                    axis=0)
            else:
                seg = x_ref[c, pl.ds(H - 3, H + 3)]
            conv = (seg[0:H] * k_ref[0]
                    + seg[1:H + 1] * k_ref[1]
                    + seg[2:H + 2] * k_ref[2]
                    + seg[3:H + 3] * k_ref[3])
            a = (conv * jax.nn.sigmoid(conv)).astype(jnp.bfloat16)
            y = jnp.dot(a, w, preferred_element_type=jnp.float32)
            out_ref[c, pl.ds(half * H, H)] = y.astype(jnp.bfloat16)

        compute_half(my, 0)
        compute_half(my, 1)

        bar = pltpu.get_barrier_semaphore()
        for nbr in (left, right):
            pl.semaphore_signal(bar, inc=1, device_id=(nbr,),
                                device_id_type=pl.DeviceIdType.MESH)
        pl.semaphore_wait(bar, 2)

        def hop(h, overlapped_compute=None):
            if h < N_DEV - 1:
                off_r, off_l = -h, h
            else:
                off_r, off_l = 4 - h, h - 4
            send_r = lax.rem(my + off_r + 2 * N_DEV, N_DEV)
            send_l = lax.rem(my + off_l + 2 * N_DEV, N_DEV)
            recv_r = lax.rem(send_r + N_DEV - 1, N_DEV)
            recv_l = lax.rem(send_l + 1, N_DEV)
            slot = h % 2

            if h >= 2:
                pl.semaphore_wait(credit_sems.at[0], 1)
                pl.semaphore_wait(credit_sems.at[1], 1)

            rdma_r = pltpu.make_async_remote_copy(
                src_ref=out_ref.at[send_r, pl.ds(0, H)],
                dst_ref=comm_r.at[slot],
                send_sem=send_sems.at[0, slot],
                recv_sem=recv_sems.at[0, slot],
                device_id=(right,),
                device_id_type=pl.DeviceIdType.MESH,
            )
            rdma_l = pltpu.make_async_remote_copy(
                src_ref=out_ref.at[send_l, pl.ds(H, H)],
                dst_ref=comm_l.at[slot],
                send_sem=send_sems.at[1, slot],
                recv_sem=recv_sems.at[1, slot],
                device_id=(left,),
                device_id_type=pl.DeviceIdType.MESH,
            )
            rdma_r.start()
            rdma_l.start()

            if overlapped_compute is not None:
                overlapped_compute()

            rdma_r.wait()
            rdma_l.wait()

            if h < N_DEV - 1:
                out_ref[recv_r, pl.ds(0, H)] = (
                    out_ref[recv_r, pl.ds(0, H)] + comm_r[slot])
                out_ref[recv_l, pl.ds(H, H)] = (
                    out_ref[recv_l, pl.ds(H, H)] + comm_l[slot])
            else:
                out_ref[recv_r, pl.ds(0, H)] = comm_r[slot]
                out_ref[recv_l, pl.ds(H, H)] = comm_l[slot]

            if h <= N_HOPS - 3:
                pl.semaphore_signal(credit_sems.at[0], inc=1,
                                    device_id=(left,),
                                    device_id_type=pl.DeviceIdType.MESH)
                pl.semaphore_signal(credit_sems.at[1], inc=1,
                                    device_id=(right,),
                                    device_id_type=pl.DeviceIdType.MESH)

        hop(0, lambda: (compute_half(cm1, 0), compute_half(cp1, 1)))
        hop(1, lambda: (compute_half(cp2, 0), compute_half(cp2, 1)))
        hop(2, lambda: (compute_half(cp1, 0), compute_half(cm1, 1)))
        hop(3)
        hop(4)
        hop(5)

    out = pl.pallas_call(
        body,
        out_shape=jax.ShapeDtypeStruct((B, S, P), jnp.bfloat16),
        in_specs=[
            pl.BlockSpec(memory_space=pltpu.VMEM),
            pl.BlockSpec(memory_space=pltpu.VMEM),
            pl.BlockSpec(memory_space=pltpu.VMEM),
        ],
        out_specs=pl.BlockSpec(memory_space=pltpu.VMEM),
        scratch_shapes=[
            pltpu.VMEM((2, H, P), jnp.bfloat16),
            pltpu.VMEM((2, H, P), jnp.bfloat16),
            pltpu.SemaphoreType.DMA((2, 2)),
            pltpu.SemaphoreType.DMA((2, 2)),
            pltpu.SemaphoreType.REGULAR((2,)),
        ],
        compiler_params=pltpu.CompilerParams(
            collective_id=0, vmem_limit_bytes=100 * 1024 * 1024),
    )(x.astype(jnp.bfloat16), k, Wp.astype(jnp.bfloat16))
    return out


# device time: 186169 ns/iter; 2.0172x vs baseline; 1.1097x over previous
import jax
import jax.numpy as jnp
from jax import lax
from jax.experimental import pallas as pl
from jax.experimental.pallas import tpu as pltpu

N_DEV = 4
N_HOPS = 2 * (N_DEV - 1)


def kernel(x, k, Wp):
    B, S, C = x.shape
    P = Wp.shape[1]
    H = S // 2

    def body(x_ref, k_ref, w_ref, out_ref, comm_r, comm_l, xstage,
             send_sems, recv_sems, stage_sems, credit_sems):
        my = lax.axis_index("i")
        left = lax.rem(my + N_DEV - 1, N_DEV)
        right = lax.rem(my + 1, N_DEV)
        cp1 = right
        cm1 = left
        cp2 = lax.rem(my + 2, N_DEV)

        w = w_ref[...]

        halves = [(my, 0), (my, 1), (cm1, 0), (cp1, 1),
                  (cp2, 0), (cp2, 1), (cp1, 0), (cm1, 1)]
        copies = [None] * len(halves)

        def start_copy(i):
            c, half = halves[i]
            if half == 0:
                cp = pltpu.make_async_copy(
                    x_ref.at[c, pl.ds(0, H)],
                    xstage.at[0, pl.ds(8, H)],
                    stage_sems.at[0])
            else:
                cp = pltpu.make_async_copy(
                    x_ref.at[c, pl.ds(H - 8, H + 8)],
                    xstage.at[1],
                    stage_sems.at[1])
            cp.start()
            copies[i] = cp

        def do_half(i):
            c, half = halves[i]
            copies[i].wait()
            seg = xstage[half, 5:]
            conv = (seg[0:H] * k_ref[0]
                    + seg[1:H + 1] * k_ref[1]
                    + seg[2:H + 2] * k_ref[2]
                    + seg[3:H + 3] * k_ref[3])
            a = (conv * jax.nn.sigmoid(conv)).astype(jnp.bfloat16)
            y = jnp.dot(a, w, preferred_element_type=jnp.float32)
            out_ref[c, pl.ds(half * H, H)] = y.astype(jnp.bfloat16)
            if i + 2 < len(halves):
                start_copy(i + 2)

        xstage[0, 5:8] = jnp.zeros((3, C), jnp.float32)
        start_copy(0)
        start_copy(1)
        do_half(0)
        do_half(1)

        bar = pltpu.get_barrier_semaphore()
        for nbr in (left, right):
            pl.semaphore_signal(bar, inc=1, device_id=(nbr,),
                                device_id_type=pl.DeviceIdType.MESH)
        pl.semaphore_wait(bar, 2)

        def hop(h, overlapped_compute=None):
            if h < N_DEV - 1:
                off_r, off_l = -h, h
            else:
                off_r, off_l = 4 - h, h - 4
            send_r = lax.rem(my + off_r + 2 * N_DEV, N_DEV)
            send_l = lax.rem(my + off_l + 2 * N_DEV, N_DEV)
            recv_r = lax.rem(send_r + N_DEV - 1, N_DEV)
            recv_l = lax.rem(send_l + 1, N_DEV)
            slot = h % 2

            if h >= 2:
                pl.semaphore_wait(credit_sems.at[0], 1)
                pl.semaphore_wait(credit_sems.at[1], 1)

            rdma_r = pltpu.make_async_remote_copy(
                src_ref=out_ref.at[send_r, pl.ds(0, H)],
                dst_ref=comm_r.at[slot],
                send_sem=send_sems.at[0, slot],
                recv_sem=recv_sems.at[0, slot],
                device_id=(right,),
                device_id_type=pl.DeviceIdType.MESH,
            )
            rdma_l = pltpu.make_async_remote_copy(
                src_ref=out_ref.at[send_l, pl.ds(H, H)],
                dst_ref=comm_l.at[slot],
                send_sem=send_sems.at[1, slot],
                recv_sem=recv_sems.at[1, slot],
                device_id=(left,),
                device_id_type=pl.DeviceIdType.MESH,
            )
            rdma_r.start()
            rdma_l.start()

            if overlapped_compute is not None:
                overlapped_compute()

            rdma_r.wait()
            rdma_l.wait()

            if h < N_DEV - 1:
                out_ref[recv_r, pl.ds(0, H)] = (
                    out_ref[recv_r, pl.ds(0, H)] + comm_r[slot])
                out_ref[recv_l, pl.ds(H, H)] = (
                    out_ref[recv_l, pl.ds(H, H)] + comm_l[slot])
            else:
                out_ref[recv_r, pl.ds(0, H)] = comm_r[slot]
                out_ref[recv_l, pl.ds(H, H)] = comm_l[slot]

            if h <= N_HOPS - 3:
                pl.semaphore_signal(credit_sems.at[0], inc=1,
                                    device_id=(left,),
                                    device_id_type=pl.DeviceIdType.MESH)
                pl.semaphore_signal(credit_sems.at[1], inc=1,
                                    device_id=(right,),
                                    device_id_type=pl.DeviceIdType.MESH)

        hop(0, lambda: (do_half(2), do_half(3)))
        hop(1, lambda: (do_half(4), do_half(5)))
        hop(2, lambda: (do_half(6), do_half(7)))
        hop(3)
        hop(4)
        hop(5)

    out = pl.pallas_call(
        body,
        out_shape=jax.ShapeDtypeStruct((B, S, P), jnp.bfloat16),
        in_specs=[
            pl.BlockSpec(memory_space=pltpu.MemorySpace.HBM),
            pl.BlockSpec(memory_space=pltpu.VMEM),
            pl.BlockSpec(memory_space=pltpu.VMEM),
        ],
        out_specs=pl.BlockSpec(memory_space=pltpu.VMEM),
        scratch_shapes=[
            pltpu.VMEM((2, H, P), jnp.bfloat16),
            pltpu.VMEM((2, H, P), jnp.bfloat16),
            pltpu.VMEM((2, H + 8, C), jnp.float32),
            pltpu.SemaphoreType.DMA((2, 2)),
            pltpu.SemaphoreType.DMA((2, 2)),
            pltpu.SemaphoreType.DMA((2,)),
            pltpu.SemaphoreType.REGULAR((2,)),
        ],
        compiler_params=pltpu.CompilerParams(
            collective_id=0, vmem_limit_bytes=100 * 1024 * 1024),
    )(x, k, Wp.astype(jnp.bfloat16))
    return out


# device time: 185983 ns/iter; 2.0193x vs baseline; 1.0010x over previous
import jax
import jax.numpy as jnp
from jax import lax
from jax.experimental import pallas as pl
from jax.experimental.pallas import tpu as pltpu

N_DEV = 4
N_HOPS = 2 * (N_DEV - 1)


def kernel(x, k, Wp):
    B, S, C = x.shape
    P = Wp.shape[1]
    H = S // 2

    def body(x_ref, k_ref, w_ref, out_ref, comm_r, comm_l, xstage,
             send_sems, recv_sems, stage_sems, credit_sems):
        my = lax.axis_index("i")
        left = lax.rem(my + N_DEV - 1, N_DEV)
        right = lax.rem(my + 1, N_DEV)
        cp1 = right
        cm1 = left
        cp2 = lax.rem(my + 2, N_DEV)

        w = w_ref[...]

        halves = [(my, 0), (my, 1), (cm1, 0), (cp1, 1),
                  (cp2, 0), (cp2, 1), (cp1, 0), (cm1, 1)]
        copies = [None] * len(halves)

        def start_copy(i):
            c, half = halves[i]
            if half == 0:
                cp = pltpu.make_async_copy(
                    x_ref.at[c, pl.ds(0, H)],
                    xstage.at[0, pl.ds(8, H)],
                    stage_sems.at[0])
            else:
                cp = pltpu.make_async_copy(
                    x_ref.at[c, pl.ds(H - 8, H + 8)],
                    xstage.at[1],
                    stage_sems.at[1])
            cp.start()
            copies[i] = cp

        def do_half(i):
            c, half = halves[i]
            copies[i].wait()
            seg = xstage[half, 5:]
            conv = (seg[0:H] * k_ref[0]
                    + seg[1:H + 1] * k_ref[1]
                    + seg[2:H + 2] * k_ref[2]
                    + seg[3:H + 3] * k_ref[3])
            a = (conv * jax.nn.sigmoid(conv)).astype(jnp.bfloat16)
            y = jnp.dot(a, w, preferred_element_type=jnp.float32)
            out_ref[c, pl.ds(half * H, H)] = y.astype(jnp.bfloat16)
            if i + 2 < len(halves):
                start_copy(i + 2)

        xstage[0, 5:8] = jnp.zeros((3, C), jnp.float32)
        start_copy(0)
        start_copy(1)

        bar = pltpu.get_barrier_semaphore()
        for nbr in (left, right):
            pl.semaphore_signal(bar, inc=1, device_id=(nbr,),
                                device_id_type=pl.DeviceIdType.MESH)
        pl.semaphore_wait(bar, 2)

        def make_rdma(ring, chunk, slot):
            if ring == 0:
                return pltpu.make_async_remote_copy(
                    src_ref=out_ref.at[chunk, pl.ds(0, H)],
                    dst_ref=comm_r.at[slot],
                    send_sem=send_sems.at[0, slot],
                    recv_sem=recv_sems.at[0, slot],
                    device_id=(right,),
                    device_id_type=pl.DeviceIdType.MESH,
                )
            return pltpu.make_async_remote_copy(
                src_ref=out_ref.at[chunk, pl.ds(H, H)],
                dst_ref=comm_l.at[slot],
                send_sem=send_sems.at[1, slot],
                recv_sem=recv_sems.at[1, slot],
                device_id=(left,),
                device_id_type=pl.DeviceIdType.MESH,
            )

        def hop(h, overlapped_compute=None, prestarted=None):
            if h < N_DEV - 1:
                off_r, off_l = -h, h
            else:
                off_r, off_l = 4 - h, h - 4
            send_r = lax.rem(my + off_r + 2 * N_DEV, N_DEV)
            send_l = lax.rem(my + off_l + 2 * N_DEV, N_DEV)
            recv_r = lax.rem(send_r + N_DEV - 1, N_DEV)
            recv_l = lax.rem(send_l + 1, N_DEV)
            slot = h % 2

            if prestarted is not None:
                rdma_r, rdma_l = prestarted
            else:
                if h >= 2:
                    pl.semaphore_wait(credit_sems.at[0], 1)
                    pl.semaphore_wait(credit_sems.at[1], 1)
                rdma_r = make_rdma(0, send_r, slot)
                rdma_l = make_rdma(1, send_l, slot)
                rdma_r.start()
                rdma_l.start()

            if overlapped_compute is not None:
                overlapped_compute()

            rdma_r.wait()
            rdma_l.wait()

            if h < N_DEV - 1:
                out_ref[recv_r, pl.ds(0, H)] = (
                    out_ref[recv_r, pl.ds(0, H)] + comm_r[slot])
                out_ref[recv_l, pl.ds(H, H)] = (
                    out_ref[recv_l, pl.ds(H, H)] + comm_l[slot])
            else:
                out_ref[recv_r, pl.ds(0, H)] = comm_r[slot]
                out_ref[recv_l, pl.ds(H, H)] = comm_l[slot]

            if h <= N_HOPS - 3:
                pl.semaphore_signal(credit_sems.at[0], inc=1,
                                    device_id=(left,),
                                    device_id_type=pl.DeviceIdType.MESH)
                pl.semaphore_signal(credit_sems.at[1], inc=1,
                                    device_id=(right,),
                                    device_id_type=pl.DeviceIdType.MESH)

        do_half(0)
        rdma_r0 = make_rdma(0, my, 0)
        rdma_r0.start()
        do_half(1)
        rdma_l0 = make_rdma(1, my, 0)
        rdma_l0.start()

        hop(0, lambda: (do_half(2), do_half(3)),
            prestarted=(rdma_r0, rdma_l0))
        hop(1, lambda: (do_half(4), do_half(5)))
        hop(2, lambda: (do_half(6), do_half(7)))
        hop(3)
        hop(4)
        hop(5)

    out = pl.pallas_call(
        body,
        out_shape=jax.ShapeDtypeStruct((B, S, P), jnp.bfloat16),
        in_specs=[
            pl.BlockSpec(memory_space=pltpu.MemorySpace.HBM),
            pl.BlockSpec(memory_space=pltpu.VMEM),
            pl.BlockSpec(memory_space=pltpu.VMEM),
        ],
        out_specs=pl.BlockSpec(memory_space=pltpu.VMEM),
        scratch_shapes=[
            pltpu.VMEM((2, H, P), jnp.bfloat16),
            pltpu.VMEM((2, H, P), jnp.bfloat16),
            pltpu.VMEM((2, H + 8, C), jnp.float32),
            pltpu.SemaphoreType.DMA((2, 2)),
            pltpu.SemaphoreType.DMA((2, 2)),
            pltpu.SemaphoreType.DMA((2,)),
            pltpu.SemaphoreType.REGULAR((2,)),
        ],
        compiler_params=pltpu.CompilerParams(
            collective_id=0, vmem_limit_bytes=100 * 1024 * 1024),
    )(x, k, Wp.astype(jnp.bfloat16))
    return out


# device time: 185296 ns/iter; 2.0267x vs baseline; 1.0037x over previous
import jax
import jax.numpy as jnp
from jax import lax
from jax.experimental import pallas as pl
from jax.experimental.pallas import tpu as pltpu

N_DEV = 4
N_HOPS = 2 * (N_DEV - 1)


def kernel(x, k, Wp):
    B, S, C = x.shape
    P = Wp.shape[1]
    H = S // 2

    def body(x_ref, k_ref, w_ref, out_ref, comm_r, comm_l, xstage,
             send_sems, recv_sems, stage_sems, credit_sems):
        my = lax.axis_index("i")
        left = lax.rem(my + N_DEV - 1, N_DEV)
        right = lax.rem(my + 1, N_DEV)
        cp1 = right
        cm1 = left
        cp2 = lax.rem(my + 2, N_DEV)

        w = w_ref[...]

        halves = [(my, 0), (my, 1), (cm1, 0), (cp1, 1),
                  (cp2, 0), (cp2, 1), (cp1, 0), (cm1, 1)]
        copies = [None] * len(halves)

        def start_copy(i):
            c, half = halves[i]
            if half == 0:
                cp = pltpu.make_async_copy(
                    x_ref.at[c, pl.ds(0, H)],
                    xstage.at[0, pl.ds(8, H)],
                    stage_sems.at[0])
            else:
                cp = pltpu.make_async_copy(
                    x_ref.at[c, pl.ds(H - 8, H + 8)],
                    xstage.at[1],
                    stage_sems.at[1])
            cp.start()
            copies[i] = cp

        def do_half(i):
            c, half = halves[i]
            copies[i].wait()
            seg = xstage[half, 5:]
            conv = (seg[0:H] * k_ref[0]
                    + seg[1:H + 1] * k_ref[1]
                    + seg[2:H + 2] * k_ref[2]
                    + seg[3:H + 3] * k_ref[3])
            a = (conv * jax.nn.sigmoid(conv)).astype(jnp.bfloat16)
            y = jnp.dot(a, w, preferred_element_type=jnp.float32)
            out_ref[c, pl.ds(half * H, H)] = y.astype(jnp.bfloat16)
            if i + 2 < len(halves):
                start_copy(i + 2)

        xstage[0, 5:8] = jnp.zeros((3, C), jnp.float32)
        start_copy(0)
        start_copy(1)

        bar = pltpu.get_barrier_semaphore()
        for nbr in (left, right):
            pl.semaphore_signal(bar, inc=1, device_id=(nbr,),
                                device_id_type=pl.DeviceIdType.MESH)
        pl.semaphore_wait(bar, 2)

        def make_rdma(ring, chunk, slot):
            if ring == 0:
                return pltpu.make_async_remote_copy(
                    src_ref=out_ref.at[chunk, pl.ds(0, H)],
                    dst_ref=comm_r.at[slot],
                    send_sem=send_sems.at[0, slot],
                    recv_sem=recv_sems.at[0, slot],
                    device_id=(right,),
                    device_id_type=pl.DeviceIdType.MESH,
                )
            return pltpu.make_async_remote_copy(
                src_ref=out_ref.at[chunk, pl.ds(H, H)],
                dst_ref=comm_l.at[slot],
                send_sem=send_sems.at[1, slot],
                recv_sem=recv_sems.at[1, slot],
                device_id=(left,),
                device_id_type=pl.DeviceIdType.MESH,
            )

        def send_chunk(ring, h):
            if h < N_DEV - 1:
                off = -h if ring == 0 else h
            else:
                off = (4 - h) if ring == 0 else (h - 4)
            return lax.rem(my + off + 2 * N_DEV, N_DEV)

        def start_hop(ring, h):
            if h >= 2:
                pl.semaphore_wait(credit_sems.at[ring], 1)
            rdma = make_rdma(ring, send_chunk(ring, h), h % 2)
            rdma.start()
            return rdma

        def finish_hop(ring, h, rdma):
            sc = send_chunk(ring, h)
            recv = lax.rem(sc + (N_DEV - 1 if ring == 0 else 1), N_DEV)
            slot = h % 2
            lo = 0 if ring == 0 else H
            comm = comm_r if ring == 0 else comm_l
            rdma.wait()
            if h < N_DEV - 1:
                out_ref[recv, pl.ds(lo, H)] = (
                    out_ref[recv, pl.ds(lo, H)] + comm[slot])
            else:
                out_ref[recv, pl.ds(lo, H)] = comm[slot]
            if h <= N_HOPS - 3:
                pl.semaphore_signal(
                    credit_sems.at[ring], inc=1,
                    device_id=(left if ring == 0 else right,),
                    device_id_type=pl.DeviceIdType.MESH)

        do_half(0)
        rR = start_hop(0, 0)
        do_half(1)
        rL = start_hop(1, 0)
        do_half(2)
        finish_hop(0, 0, rR)
        rR = start_hop(0, 1)
        do_half(3)
        finish_hop(1, 0, rL)
        rL = start_hop(1, 1)
        do_half(4)
        finish_hop(0, 1, rR)
        rR = start_hop(0, 2)
        do_half(5)
        finish_hop(1, 1, rL)
        rL = start_hop(1, 2)
        do_half(6)
        finish_hop(0, 2, rR)
        rR = start_hop(0, 3)
        do_half(7)
        finish_hop(1, 2, rL)
        rL = start_hop(1, 3)
        for h in range(3, N_HOPS):
            finish_hop(0, h, rR)
            rR = start_hop(0, h + 1) if h + 1 < N_HOPS else None
            finish_hop(1, h, rL)
            rL = start_hop(1, h + 1) if h + 1 < N_HOPS else None

    out = pl.pallas_call(
        body,
        out_shape=jax.ShapeDtypeStruct((B, S, P), jnp.bfloat16),
        in_specs=[
            pl.BlockSpec(memory_space=pltpu.MemorySpace.HBM),
            pl.BlockSpec(memory_space=pltpu.VMEM),
            pl.BlockSpec(memory_space=pltpu.VMEM),
        ],
        out_specs=pl.BlockSpec(memory_space=pltpu.VMEM),
        scratch_shapes=[
            pltpu.VMEM((2, H, P), jnp.bfloat16),
            pltpu.VMEM((2, H, P), jnp.bfloat16),
            pltpu.VMEM((2, H + 8, C), jnp.float32),
            pltpu.SemaphoreType.DMA((2, 2)),
            pltpu.SemaphoreType.DMA((2, 2)),
            pltpu.SemaphoreType.DMA((2,)),
            pltpu.SemaphoreType.REGULAR((2,)),
        ],
        compiler_params=pltpu.CompilerParams(
            collective_id=0, vmem_limit_bytes=100 * 1024 * 1024),
    )(x, k, Wp.astype(jnp.bfloat16))
    return out
